# baseline (speedup 1.0000x reference)
import functools

import jax
import jax.numpy as jnp
from jax import lax
from jax.experimental import pallas as pl
from jax.experimental.pallas import tpu as pltpu
from jax.experimental.pallas import tpu_sc as plsc

_B, _N, _F, _K = 32, 8192, 16, 1024
_BB = 8
_EPS = 1e-10
_IMIN = -2147483648


def _min_kernel(x_ref, o_ref):
    b = pl.program_id(0)
    m = jnp.full((1, 1), jnp.min(jnp.abs(x_ref[...])), jnp.float32)

    @pl.when(b == 0)
    def _():
        o_ref[:, :] = m

    @pl.when(b != 0)
    def _():
        o_ref[:, :] = jnp.minimum(o_ref[:, :], m)


def _select_kernel(xt_ref, m0t_ref, minv_ref, mask_ref):
    x = xt_ref[...]
    m0 = m0t_ref[...]
    v = (x + (minv_ref[:, :] + _EPS)[:, :, None]) * m0
    bits = jax.lax.bitcast_convert_type(v, jnp.int32)
    keys = jnp.where(bits < 0, bits ^ jnp.int32(0x7FFFFFFF), bits)

    def vcond(ipc):
        i, _, c = ipc
        return jnp.logical_and(i < 32, jnp.logical_not(jnp.all(c == _K)))

    def vbody(ipc):
        i, p, c = ipc
        cand = p | jax.lax.shift_left(jnp.int32(1), 31 - i)
        cnt = jnp.sum((keys >= (cand ^ jnp.int32(_IMIN))).astype(jnp.int32),
                      axis=2, keepdims=True)
        acc = cnt >= _K
        return i + 1, jnp.where(acc, cand, p), jnp.where(acc, cnt, c)

    _, p, c = jax.lax.while_loop(
        vcond, vbody, (jnp.int32(0), jnp.zeros((_BB, _F, 1), jnp.int32),
                       jnp.full((_BB, _F, 1), _N, jnp.int32)))
    tkey = p ^ jnp.int32(_IMIN)

    no_ties = jnp.all(c == _K)

    @pl.when(no_ties)
    def _():
        sel = keys >= tkey
        maskf = jnp.max(sel.astype(jnp.float32), axis=1, keepdims=True)
        mask_ref[...] = maskf

    @pl.when(jnp.logical_not(no_ties))
    def _():
        gt = keys > tkey
        eq = keys == tkey
        iota = jax.lax.broadcasted_iota(jnp.int32, (_BB, _F, _N), 2)
        iota_m = jnp.where(eq, iota, jnp.int32(_N))
        g0 = jnp.sum(gt.astype(jnp.int32), axis=2, keepdims=True)

        def ibody(i, p2):
            cand = p2 | jax.lax.shift_left(jnp.int32(1), 12 - i)
            cnt = g0 + jnp.sum((iota_m <= cand).astype(jnp.int32), axis=2,
                               keepdims=True)
            return jnp.where(cnt < _K, cand, p2)

        p2 = jax.lax.fori_loop(0, 13, ibody,
                               jnp.zeros((_BB, _F, 1), jnp.int32))
        gp = g0 + jnp.sum((iota_m <= p2).astype(jnp.int32), axis=2,
                          keepdims=True)
        jstar = p2 + (gp < _K).astype(jnp.int32)

        sel = gt | (iota_m <= jstar)
        maskf = jnp.max(sel.astype(jnp.float32), axis=1, keepdims=True)
        mask_ref[...] = maskf


_CH = 1024


def _make_sc_mul():
    mesh = plsc.VectorSubcoreMesh(core_axis_name="c", subcore_axis_name="s")

    @functools.partial(
        pl.kernel, mesh=mesh,
        out_type=jax.ShapeDtypeStruct((_B * _N * _F,), jnp.float32),
        scratch_types=[
            pltpu.VMEM((_N,), jnp.float32),
            pltpu.VMEM((_CH * _F,), jnp.float32),
        ],
    )
    def _sc_mul(x_hbm, m_hbm, out_hbm, mask_v, buf_v):
        wid = lax.axis_index("s") * 2 + lax.axis_index("c")
        pltpu.sync_copy(m_hbm.at[pl.ds(wid * _N, _N)], mask_v)

        def chunk(ci, carry):
            base = ci * _CH
            pltpu.sync_copy(
                x_hbm.at[pl.ds((wid * _N + base) * _F, _CH * _F)], buf_v)

            def row16(r, c2):
                m16 = mask_v[pl.ds(base + r * 16, 16)]
                for j in range(16):
                    mj = lax.broadcast_in_dim(
                        lax.slice(m16, (j,), (j + 1,)), (16,), (0,))
                    off = (r * 16 + j) * _F
                    buf_v[pl.ds(off, _F)] = buf_v[pl.ds(off, _F)] * mj
                return c2

            lax.fori_loop(0, _CH // 16, row16, 0)
            pltpu.sync_copy(
                buf_v, out_hbm.at[pl.ds((wid * _N + base) * _F, _CH * _F)])
            return carry

        lax.fori_loop(0, _N // _CH, chunk, 0)

    return _sc_mul


_sc_mul_kernel = _make_sc_mul()


@jax.jit
def kernel(input, mask, init_mask):
    del mask
    xt = jnp.transpose(input, (0, 2, 1))
    m0t = jnp.transpose(init_mask, (0, 2, 1))

    minv = pl.pallas_call(
        _min_kernel,
        grid=(_B // _BB,),
        in_specs=[pl.BlockSpec((_BB, _F, _N), lambda b: (b, 0, 0))],
        out_specs=pl.BlockSpec((1, 1), lambda b: (0, 0)),
        out_shape=jax.ShapeDtypeStruct((1, 1), jnp.float32),
    )(xt)

    mask_t = pl.pallas_call(
        _select_kernel,
        grid=(_B // _BB,),
        in_specs=[
            pl.BlockSpec((_BB, _F, _N), lambda b: (b, 0, 0)),
            pl.BlockSpec((_BB, 1, _N), lambda b: (b, 0, 0)),
            pl.BlockSpec((1, 1), lambda b: (0, 0)),
        ],
        out_specs=pl.BlockSpec((_BB, 1, _N), lambda b: (b, 0, 0)),
        out_shape=jax.ShapeDtypeStruct((_B, 1, _N), jnp.float32),
    )(xt, m0t, minv)

    out2 = _sc_mul_kernel(jnp.reshape(input, (_B * _N * _F,)),
                          jnp.reshape(mask_t, (_B * _N,)))
    updated_mask = jnp.reshape(mask_t, (_B, _N, 1))
    masked_out = jnp.reshape(out2, (_B, _N, _F))
    return (updated_mask, masked_out)


# speedup vs baseline: 3.7475x; 3.7475x over previous
import jax
import jax.numpy as jnp
from jax.experimental import pallas as pl

_B, _N, _F, _K = 32, 8192, 16, 1024
_BB = 8
_EPS = 1e-10
_IMIN = -2147483648


def _min_kernel(x_ref, o_ref):
    b = pl.program_id(0)
    m = jnp.full((1, 1), jnp.min(jnp.abs(x_ref[...])), jnp.float32)

    @pl.when(b == 0)
    def _():
        o_ref[:, :] = m

    @pl.when(b != 0)
    def _():
        o_ref[:, :] = jnp.minimum(o_ref[:, :], m)


def _select_kernel(xt_ref, m0t_ref, minv_ref, out_ref, mask_ref):
    x = xt_ref[...]
    m0 = m0t_ref[...]
    v = (x + (minv_ref[:, :] + _EPS)[:, :, None]) * m0
    bits = jax.lax.bitcast_convert_type(v, jnp.int32)
    keys = jnp.where(bits < 0, bits ^ jnp.int32(0x7FFFFFFF), bits)

    cnt0 = jnp.sum((keys >= 0).astype(jnp.int32), axis=2, keepdims=True)
    acc0 = cnt0 >= _K
    p0 = jnp.where(acc0, jnp.int32(_IMIN), jnp.int32(0))
    c0 = jnp.where(acc0, cnt0, jnp.int32(_N))

    def vcond(ipc):
        i, _, c = ipc
        return jnp.logical_and(i < 32, jnp.logical_not(jnp.all(c == _K)))

    def vbody(ipc):
        i, p, c = ipc
        cand = p | jax.lax.shift_left(jnp.int32(1), 31 - i)
        cnt = jnp.sum((keys >= (cand ^ jnp.int32(_IMIN))).astype(jnp.int32),
                      axis=2, keepdims=True)
        acc = cnt >= _K
        return i + 1, jnp.where(acc, cand, p), jnp.where(acc, cnt, c)

    _, p, c = jax.lax.while_loop(vcond, vbody, (jnp.int32(1), p0, c0))
    tkey = p ^ jnp.int32(_IMIN)

    no_ties = jnp.all(c == _K)

    @pl.when(no_ties)
    def _():
        sel = keys >= tkey
        maskf = jnp.max(sel.astype(jnp.float32), axis=1, keepdims=True)
        mask_ref[...] = maskf
        out_ref[...] = x * maskf

    @pl.when(jnp.logical_not(no_ties))
    def _():
        gt = keys > tkey
        eq = keys == tkey
        iota = jax.lax.broadcasted_iota(jnp.int32, (_BB, _F, _N), 2)
        iota_m = jnp.where(eq, iota, jnp.int32(_N))
        g0 = jnp.sum(gt.astype(jnp.int32), axis=2, keepdims=True)

        def ibody(i, p2):
            cand = p2 | jax.lax.shift_left(jnp.int32(1), 12 - i)
            cnt = g0 + jnp.sum((iota_m <= cand).astype(jnp.int32), axis=2,
                               keepdims=True)
            return jnp.where(cnt < _K, cand, p2)

        p2 = jax.lax.fori_loop(0, 13, ibody,
                               jnp.zeros((_BB, _F, 1), jnp.int32))
        gp = g0 + jnp.sum((iota_m <= p2).astype(jnp.int32), axis=2,
                          keepdims=True)
        jstar = p2 + (gp < _K).astype(jnp.int32)

        sel = gt | (iota_m <= jstar)
        maskf = jnp.max(sel.astype(jnp.float32), axis=1, keepdims=True)
        mask_ref[...] = maskf
        out_ref[...] = x * maskf


@jax.jit
def kernel(input, mask, init_mask):
    del mask
    xt = jnp.transpose(input, (0, 2, 1))
    m0t = jnp.transpose(init_mask, (0, 2, 1))

    minv = pl.pallas_call(
        _min_kernel,
        grid=(_B // _BB,),
        in_specs=[pl.BlockSpec((_BB, _F, _N), lambda b: (b, 0, 0))],
        out_specs=pl.BlockSpec((1, 1), lambda b: (0, 0)),
        out_shape=jax.ShapeDtypeStruct((1, 1), jnp.float32),
    )(xt)

    out_t, mask_t = pl.pallas_call(
        _select_kernel,
        grid=(_B // _BB,),
        in_specs=[
            pl.BlockSpec((_BB, _F, _N), lambda b: (b, 0, 0)),
            pl.BlockSpec((_BB, 1, _N), lambda b: (b, 0, 0)),
            pl.BlockSpec((1, 1), lambda b: (0, 0)),
        ],
        out_specs=[
            pl.BlockSpec((_BB, _F, _N), lambda b: (b, 0, 0)),
            pl.BlockSpec((_BB, 1, _N), lambda b: (b, 0, 0)),
        ],
        out_shape=[
            jax.ShapeDtypeStruct((_B, _F, _N), jnp.float32),
            jax.ShapeDtypeStruct((_B, 1, _N), jnp.float32),
        ],
    )(xt, m0t, minv)

    updated_mask = jnp.reshape(mask_t, (_B, _N, 1))
    masked_out = jnp.transpose(out_t, (0, 2, 1))
    return (updated_mask, masked_out)


# speedup vs baseline: 3.8138x; 1.0177x over previous
import jax
import jax.numpy as jnp
from jax.experimental import pallas as pl

_B, _N, _F, _K = 32, 8192, 16, 1024
_BB = 8
_EPS = 1e-10
_IMIN = -2147483648


def _min_kernel(x_ref, o_ref):
    b = pl.program_id(0)
    m = jnp.full((1, 1), jnp.min(jnp.abs(x_ref[...])), jnp.float32)

    @pl.when(b == 0)
    def _():
        o_ref[:, :] = m

    @pl.when(b != 0)
    def _():
        o_ref[:, :] = jnp.minimum(o_ref[:, :], m)


def _select_kernel(xt_ref, m0t_ref, minv_ref, out_ref, mask_ref):
    x = xt_ref[...]
    m0 = m0t_ref[...]
    v = (x + (minv_ref[:, :] + _EPS)[:, :, None]) * m0
    bits = jax.lax.bitcast_convert_type(v, jnp.int32)
    keys = jnp.where(bits < 0, bits ^ jnp.int32(0x7FFFFFFF), bits)

    cnt0 = jnp.sum((keys >= 0).astype(jnp.int32), axis=2, keepdims=True)
    acc0 = cnt0 >= _K
    p0 = jnp.where(acc0, jnp.int32(_IMIN), jnp.int32(0))
    c0 = jnp.where(acc0, cnt0, jnp.int32(_N))

    def vcond(ipc):
        i, _, c = ipc
        return jnp.logical_and(i < 32, jnp.logical_not(jnp.all(c == _K)))

    def vstep(i, p, c):
        cand = p | jax.lax.shift_left(jnp.int32(1),
                                      jnp.maximum(31 - i, jnp.int32(0)))
        cnt = jnp.sum((keys >= (cand ^ jnp.int32(_IMIN))).astype(jnp.int32),
                      axis=2, keepdims=True)
        acc = cnt >= _K
        return jnp.where(acc, cand, p), jnp.where(acc, cnt, c)

    def vbody(ipc):
        i, p, c = ipc
        p, c = vstep(i, p, c)
        p, c = vstep(i + 1, p, c)
        return i + 2, p, c

    _, p, c = jax.lax.while_loop(vcond, vbody, (jnp.int32(1), p0, c0))
    tkey = p ^ jnp.int32(_IMIN)

    no_ties = jnp.all(c == _K)

    @pl.when(no_ties)
    def _():
        sel = keys >= tkey
        maskf = jnp.max(sel.astype(jnp.float32), axis=1, keepdims=True)
        mask_ref[...] = maskf
        out_ref[...] = x * maskf

    @pl.when(jnp.logical_not(no_ties))
    def _():
        gt = keys > tkey
        eq = keys == tkey
        iota = jax.lax.broadcasted_iota(jnp.int32, (_BB, _F, _N), 2)
        iota_m = jnp.where(eq, iota, jnp.int32(_N))
        g0 = jnp.sum(gt.astype(jnp.int32), axis=2, keepdims=True)

        def ibody(i, p2):
            cand = p2 | jax.lax.shift_left(jnp.int32(1), 12 - i)
            cnt = g0 + jnp.sum((iota_m <= cand).astype(jnp.int32), axis=2,
                               keepdims=True)
            return jnp.where(cnt < _K, cand, p2)

        p2 = jax.lax.fori_loop(0, 13, ibody,
                               jnp.zeros((_BB, _F, 1), jnp.int32))
        gp = g0 + jnp.sum((iota_m <= p2).astype(jnp.int32), axis=2,
                          keepdims=True)
        jstar = p2 + (gp < _K).astype(jnp.int32)

        sel = gt | (iota_m <= jstar)
        maskf = jnp.max(sel.astype(jnp.float32), axis=1, keepdims=True)
        mask_ref[...] = maskf
        out_ref[...] = x * maskf


@jax.jit
def kernel(input, mask, init_mask):
    del mask
    xt = jnp.transpose(input, (0, 2, 1))
    m0t = jnp.transpose(init_mask, (0, 2, 1))

    minv = pl.pallas_call(
        _min_kernel,
        grid=(_B // _BB,),
        in_specs=[pl.BlockSpec((_BB, _F, _N), lambda b: (b, 0, 0))],
        out_specs=pl.BlockSpec((1, 1), lambda b: (0, 0)),
        out_shape=jax.ShapeDtypeStruct((1, 1), jnp.float32),
    )(xt)

    out_t, mask_t = pl.pallas_call(
        _select_kernel,
        grid=(_B // _BB,),
        in_specs=[
            pl.BlockSpec((_BB, _F, _N), lambda b: (b, 0, 0)),
            pl.BlockSpec((_BB, 1, _N), lambda b: (b, 0, 0)),
            pl.BlockSpec((1, 1), lambda b: (0, 0)),
        ],
        out_specs=[
            pl.BlockSpec((_BB, _F, _N), lambda b: (b, 0, 0)),
            pl.BlockSpec((_BB, 1, _N), lambda b: (b, 0, 0)),
        ],
        out_shape=[
            jax.ShapeDtypeStruct((_B, _F, _N), jnp.float32),
            jax.ShapeDtypeStruct((_B, 1, _N), jnp.float32),
        ],
    )(xt, m0t, minv)

    updated_mask = jnp.reshape(mask_t, (_B, _N, 1))
    masked_out = jnp.transpose(out_t, (0, 2, 1))
    return (updated_mask, masked_out)
